# Initial kernel scaffold; baseline (speedup 1.0000x reference)
import jax
import jax.numpy as jnp
from jax.experimental import pallas as pl


def kernel(idx, wte, wpe):
    raise NotImplementedError("write your pallas kernel here")


# speedup vs baseline: 1.0572x; 1.0572x over previous
import jax
import jax.numpy as jnp
from jax import lax
from jax.experimental import pallas as pl
from jax.experimental.pallas import tpu as pltpu
from jax.experimental.pallas import tpu_sc as plsc

NC = 2
NS = 16
NW = NC * NS
B = 16
T = 2048
D = 128
B_TOT = B * T
PER_W = B_TOT // NW
C = 256
NCHUNK = PER_W // C


def _embd_body(wte_hbm, idx_hbm, wpe_hbm, out_hbm, idx_v, buf, sem):
    cid = lax.axis_index("c")
    sid = lax.axis_index("s")
    wid = sid * NC + cid
    base = wid * PER_W
    pos_base = lax.rem(base, T)
    pltpu.sync_copy(idx_hbm.at[pl.ds(base, PER_W)], idx_v)
    for c in range(NCHUNK):
        pltpu.sync_copy(wpe_hbm.at[pl.ds(pos_base + c * C, C)], buf)
        pltpu.async_copy(
            wte_hbm.at[idx_v.at[pl.ds(c * C, C)]], buf, sem, add=True
        ).wait()
        pltpu.sync_copy(buf, out_hbm.at[pl.ds(base + c * C, C)])


def kernel(idx, wte, wpe):
    idx_flat = idx.reshape(-1).astype(jnp.int32)
    run = pl.kernel(
        _embd_body,
        out_type=jax.ShapeDtypeStruct((B_TOT, D), jnp.float32),
        mesh=plsc.VectorSubcoreMesh(core_axis_name="c", subcore_axis_name="s"),
        scratch_types=[
            pltpu.VMEM((PER_W,), jnp.int32),
            pltpu.VMEM((C, D), jnp.float32),
            pltpu.SemaphoreType.DMA,
        ],
    )
    out = run(wte, idx_flat, wpe)
    return out.reshape(B, T, D)


# speedup vs baseline: 1.0990x; 1.0396x over previous
import jax
import jax.numpy as jnp
from jax import lax
from jax.experimental import pallas as pl
from jax.experimental.pallas import tpu as pltpu
from jax.experimental.pallas import tpu_sc as plsc

NC = 2
NS = 16
NW = NC * NS
B = 16
T = 2048
D = 128
B_TOT = B * T
PER_W = B_TOT // NW
C = 128
NCHUNK = PER_W // C
NBUF = 4


def _embd_body(wte_hbm, idx_hbm, wpe_hbm, out_hbm, idx_v, *rest):
    bufs = rest[:NBUF]
    s_w = rest[NBUF:2 * NBUF]
    s_g = rest[2 * NBUF:3 * NBUF]
    s_o = rest[3 * NBUF:4 * NBUF]

    cid = lax.axis_index("c")
    sid = lax.axis_index("s")
    wid = sid * NC + cid
    base = wid * PER_W
    pos_base = lax.rem(base, T)

    pltpu.sync_copy(idx_hbm.at[pl.ds(base, PER_W)], idx_v)

    w_cp = [None] * NCHUNK
    o_cp = [None] * NCHUNK
    for c in range(NBUF):
        w_cp[c] = pltpu.async_copy(
            wpe_hbm.at[pl.ds(pos_base + c * C, C)], bufs[c], s_w[c]
        )
    for c in range(NCHUNK):
        b = bufs[c % NBUF]
        w_cp[c].wait()
        g = pltpu.async_copy(
            wte_hbm.at[idx_v.at[pl.ds(c * C, C)]], b, s_g[c % NBUF], add=True
        )
        n = c - 1 + NBUF
        if c >= 1 and n < NCHUNK:
            o_cp[c - 1].wait()
            w_cp[n] = pltpu.async_copy(
                wpe_hbm.at[pl.ds(pos_base + n * C, C)],
                bufs[n % NBUF],
                s_w[n % NBUF],
            )
        g.wait()
        o_cp[c] = pltpu.async_copy(
            b, out_hbm.at[pl.ds(base + c * C, C)], s_o[c % NBUF]
        )
    for c in range(NCHUNK - NBUF, NCHUNK):
        o_cp[c].wait()


def kernel(idx, wte, wpe):
    idx_flat = idx.reshape(-1).astype(jnp.int32)
    run = pl.kernel(
        _embd_body,
        out_type=jax.ShapeDtypeStruct((B_TOT, D), jnp.float32),
        mesh=plsc.VectorSubcoreMesh(core_axis_name="c", subcore_axis_name="s"),
        scratch_types=(
            [pltpu.VMEM((PER_W,), jnp.int32)]
            + [pltpu.VMEM((C, D), jnp.float32) for _ in range(NBUF)]
            + [pltpu.SemaphoreType.DMA for _ in range(3 * NBUF)]
        ),
    )
    out = run(wte, idx_flat, wpe)
    return out.reshape(B, T, D)


# speedup vs baseline: 1.3232x; 1.2040x over previous
import jax
import jax.numpy as jnp
from jax import lax
from jax.experimental import pallas as pl
from jax.experimental.pallas import tpu as pltpu
from jax.experimental.pallas import tpu_sc as plsc

NC = 2
NS = 16
NW = NC * NS
B = 16
T = 2048
D = 128
B_TOT = B * T
PER_W = B_TOT // NW
C = 128
NCHUNK = PER_W // C
NBUF = 4


def _embd_body(wte_hbm, idx_hbm, wpe_hbm, out_hbm, idx_v, wpe_sh, *rest):
    bufs = rest[:NBUF]
    s_w = rest[NBUF:2 * NBUF]
    s_g = rest[2 * NBUF:3 * NBUF]
    s_o = rest[3 * NBUF:4 * NBUF]

    cid = lax.axis_index("c")
    sid = lax.axis_index("s")
    wid = sid * NC + cid
    base = wid * PER_W
    pos_base = cid * PER_W

    @pl.when(sid == 0)
    def _stage():
        pltpu.sync_copy(wpe_hbm.at[pl.ds(pos_base, PER_W)], wpe_sh)

    pltpu.sync_copy(idx_hbm.at[pl.ds(base, PER_W)], idx_v)
    plsc.subcore_barrier()

    w_cp = [None] * NCHUNK
    o_cp = [None] * NCHUNK
    for c in range(NBUF):
        w_cp[c] = pltpu.async_copy(
            wpe_sh.at[pl.ds(c * C, C)], bufs[c], s_w[c]
        )
    for c in range(NCHUNK):
        b = bufs[c % NBUF]
        w_cp[c].wait()
        g = pltpu.async_copy(
            wte_hbm.at[idx_v.at[pl.ds(c * C, C)]], b, s_g[c % NBUF], add=True
        )
        n = c - 1 + NBUF
        if c >= 1 and n < NCHUNK:
            o_cp[c - 1].wait()
            w_cp[n] = pltpu.async_copy(
                wpe_sh.at[pl.ds(n * C, C)],
                bufs[n % NBUF],
                s_w[n % NBUF],
            )
        g.wait()
        o_cp[c] = pltpu.async_copy(
            b, out_hbm.at[pl.ds(base + c * C, C)], s_o[c % NBUF]
        )
    for c in range(NCHUNK - NBUF, NCHUNK):
        o_cp[c].wait()


def kernel(idx, wte, wpe):
    idx_flat = idx.reshape(-1).astype(jnp.int32)
    run = pl.kernel(
        _embd_body,
        out_type=jax.ShapeDtypeStruct((B_TOT, D), jnp.float32),
        mesh=plsc.VectorSubcoreMesh(core_axis_name="c", subcore_axis_name="s"),
        scratch_types=(
            [pltpu.VMEM((PER_W,), jnp.int32),
             pltpu.VMEM_SHARED((PER_W, D), jnp.float32)]
            + [pltpu.VMEM((C, D), jnp.float32) for _ in range(NBUF)]
            + [pltpu.SemaphoreType.DMA for _ in range(3 * NBUF)]
        ),
    )
    out = run(wte, idx_flat, wpe)
    return out.reshape(B, T, D)


# speedup vs baseline: 1.4386x; 1.0872x over previous
import jax
import jax.numpy as jnp
from jax import lax
from jax.experimental import pallas as pl
from jax.experimental.pallas import tpu as pltpu
from jax.experimental.pallas import tpu_sc as plsc

NC = 2
NS = 16
NW = NC * NS
B = 16
T = 2048
D = 128
B_TOT = B * T
PER_W = B_TOT // NW
C = 128
NCHUNK = PER_W // C
NBUF = 6
OLAG = 2


def _embd_body(wte_hbm, idx_hbm, wpe_hbm, out_hbm, idx_v, wpe_sh, *rest):
    bufs = rest[:NBUF]
    s_w = rest[NBUF:2 * NBUF]
    s_g = rest[2 * NBUF:3 * NBUF]
    s_o = rest[3 * NBUF:4 * NBUF]

    cid = lax.axis_index("c")
    sid = lax.axis_index("s")
    wid = sid * NC + cid
    base = wid * PER_W
    pos_base = cid * PER_W

    @pl.when(sid == 0)
    def _stage():
        pltpu.sync_copy(wpe_hbm.at[pl.ds(pos_base, PER_W)], wpe_sh)

    pltpu.sync_copy(idx_hbm.at[pl.ds(base, PER_W)], idx_v)
    plsc.subcore_barrier()

    w_cp = [None] * NCHUNK
    g_cp = [None] * NCHUNK
    o_cp = [None] * NCHUNK
    o_waited = [False] * NCHUNK
    for c in range(min(NBUF, NCHUNK)):
        w_cp[c] = pltpu.async_copy(
            wpe_sh.at[pl.ds(c * C, C)], bufs[c], s_w[c]
        )
    for step in range(NCHUNK + OLAG):
        c = step
        if c < NCHUNK:
            w_cp[c].wait()
            g_cp[c] = pltpu.async_copy(
                wte_hbm.at[idx_v.at[pl.ds(c * C, C)]],
                bufs[c % NBUF],
                s_g[c % NBUF],
                add=True,
            )
        co = step - OLAG
        if 0 <= co < NCHUNK:
            g_cp[co].wait()
            o_cp[co] = pltpu.async_copy(
                bufs[co % NBUF],
                out_hbm.at[pl.ds(base + co * C, C)],
                s_o[co % NBUF],
            )
        n = step + NBUF - OLAG - 1
        if NBUF <= n < NCHUNK:
            o_cp[n - NBUF].wait()
            o_waited[n - NBUF] = True
            w_cp[n] = pltpu.async_copy(
                wpe_sh.at[pl.ds(n * C, C)],
                bufs[n % NBUF],
                s_w[n % NBUF],
            )
    for c in range(NCHUNK):
        if not o_waited[c]:
            o_cp[c].wait()


def kernel(idx, wte, wpe):
    idx_flat = idx.reshape(-1).astype(jnp.int32)
    run = pl.kernel(
        _embd_body,
        out_type=jax.ShapeDtypeStruct((B_TOT, D), jnp.float32),
        mesh=plsc.VectorSubcoreMesh(core_axis_name="c", subcore_axis_name="s"),
        scratch_types=(
            [pltpu.VMEM((PER_W,), jnp.int32),
             pltpu.VMEM_SHARED((PER_W, D), jnp.float32)]
            + [pltpu.VMEM((C, D), jnp.float32) for _ in range(NBUF)]
            + [pltpu.SemaphoreType.DMA for _ in range(3 * NBUF)]
        ),
    )
    out = run(wte, idx_flat, wpe)
    return out.reshape(B, T, D)


# speedup vs baseline: 1.4616x; 1.0160x over previous
import jax
import jax.numpy as jnp
from jax import lax
from jax.experimental import pallas as pl
from jax.experimental.pallas import tpu as pltpu
from jax.experimental.pallas import tpu_sc as plsc

NC = 2
NS = 16
NW = NC * NS
B = 16
T = 2048
D = 128
B_TOT = B * T
PER_W = B_TOT // NW
C = 128
NCHUNK = PER_W // C
NBUF = 6
OLAG = 3


def _embd_body(wte_hbm, idx_hbm, wpe_hbm, out_hbm, idx_v, wpe_sh, *rest):
    bufs = rest[:NBUF]
    s_w = rest[NBUF:2 * NBUF]
    s_g = rest[2 * NBUF:3 * NBUF]
    s_o = rest[3 * NBUF:4 * NBUF]

    cid = lax.axis_index("c")
    sid = lax.axis_index("s")
    wid = sid * NC + cid
    base = wid * PER_W
    pos_base = cid * PER_W

    @pl.when(sid == 0)
    def _stage():
        pltpu.sync_copy(wpe_hbm.at[pl.ds(pos_base, PER_W)], wpe_sh)

    pltpu.sync_copy(idx_hbm.at[pl.ds(base, PER_W)], idx_v)
    plsc.subcore_barrier()

    w_cp = [None] * NCHUNK
    g_cp = [None] * NCHUNK
    o_cp = [None] * NCHUNK
    o_waited = [False] * NCHUNK
    for c in range(min(NBUF, NCHUNK)):
        w_cp[c] = pltpu.async_copy(
            wpe_sh.at[pl.ds(c * C, C)], bufs[c], s_w[c]
        )
    for step in range(NCHUNK + OLAG):
        c = step
        if c < NCHUNK:
            w_cp[c].wait()
            g_cp[c] = pltpu.async_copy(
                wte_hbm.at[idx_v.at[pl.ds(c * C, C)]],
                bufs[c % NBUF],
                s_g[c % NBUF],
                add=True,
            )
        co = step - OLAG
        if 0 <= co < NCHUNK:
            g_cp[co].wait()
            o_cp[co] = pltpu.async_copy(
                bufs[co % NBUF],
                out_hbm.at[pl.ds(base + co * C, C)],
                s_o[co % NBUF],
            )
        n = step + NBUF - OLAG - 1
        if NBUF <= n < NCHUNK:
            o_cp[n - NBUF].wait()
            o_waited[n - NBUF] = True
            w_cp[n] = pltpu.async_copy(
                wpe_sh.at[pl.ds(n * C, C)],
                bufs[n % NBUF],
                s_w[n % NBUF],
            )
    for c in range(NCHUNK):
        if not o_waited[c]:
            o_cp[c].wait()


def kernel(idx, wte, wpe):
    idx_flat = idx.reshape(-1).astype(jnp.int32)
    run = pl.kernel(
        _embd_body,
        out_type=jax.ShapeDtypeStruct((B_TOT, D), jnp.float32),
        mesh=plsc.VectorSubcoreMesh(core_axis_name="c", subcore_axis_name="s"),
        scratch_types=(
            [pltpu.VMEM((PER_W,), jnp.int32),
             pltpu.VMEM_SHARED((PER_W, D), jnp.float32)]
            + [pltpu.VMEM((C, D), jnp.float32) for _ in range(NBUF)]
            + [pltpu.SemaphoreType.DMA for _ in range(3 * NBUF)]
        ),
    )
    out = run(wte, idx_flat, wpe)
    return out.reshape(B, T, D)
